# baseline (device time: 474767 ns/iter reference)
import jax
import jax.numpy as jnp
from jax import lax
from jax.experimental import pallas as pl
from jax.experimental.pallas import tpu as pltpu

N_DEV = 8
HPD = 8
DH = 128
NG = 4
SCALE = 0.08838834764831843


def _group_rows(a):
    n, d = a.shape
    return a.reshape(n // 256, NG, 64, d).transpose(1, 0, 2, 3).reshape(n, d)


def _body(x_ref, k_ref, v_ref, w_ref, out_ref,
          comm_ref, kbuf, vbuf, send_sems, recv_sems, kcp_sems, vcp_sems):
    my = lax.axis_index("i")
    left = jnp.mod(my - 1, N_DEV)
    right = jnp.mod(my + 1, N_DEV)

    def kv_copies(h):
        j = jnp.mod(my - h, N_DEV)
        slot = h % 2
        kcp = pltpu.make_async_copy(
            k_ref.at[pl.ds(j * HPD, HPD)], kbuf.at[slot], kcp_sems.at[slot])
        vcp = pltpu.make_async_copy(
            v_ref.at[pl.ds(j * HPD, HPD)], vbuf.at[slot], vcp_sems.at[slot])
        return kcp, vcp

    for cp in kv_copies(0):
        cp.start()

    barrier = pltpu.get_barrier_semaphore()
    for nbr in (left, right):
        pl.semaphore_signal(barrier, inc=1, device_id=(nbr,),
                            device_id_type=pl.DeviceIdType.MESH)
    pl.semaphore_wait(barrier, 2)

    comm_ref[0] = w_ref[...]

    x_bf = x_ref[...]
    sq = x_bf.shape[0]

    def mm(a, b, dims):
        return lax.dot_general(a, b, dimension_numbers=(dims, ((), ())),
                               preferred_element_type=jnp.float32)

    acc = jnp.zeros(out_ref.shape, jnp.float32)

    for h in range(N_DEV):
        send_slot = h % 2
        recv_slot = (h + 1) % 2
        if h < N_DEV - 1:
            rdma = pltpu.make_async_remote_copy(
                src_ref=comm_ref.at[send_slot],
                dst_ref=comm_ref.at[recv_slot],
                send_sem=send_sems.at[send_slot],
                recv_sem=recv_sems.at[recv_slot],
                device_id=(right,),
                device_id_type=pl.DeviceIdType.MESH,
            )
            rdma.start()
            for cp in kv_copies(h + 1):
                cp.start()

        for cp in kv_copies(h):
            cp.wait()

        def head_body(g, acc, slot=send_slot):
            wq_g = comm_ref[slot, 2 * g]
            wo_g = comm_ref[slot, 2 * g + 1]
            kh = kbuf[slot, g].reshape(NG, 256, DH)
            vh = vbuf[slot, g].reshape(NG, 256, DH)
            qh = mm(x_bf, wq_g, ((1,), (0,))).astype(jnp.bfloat16)
            q3 = qh.reshape(NG, 256, DH)
            scores = lax.dot_general(
                q3, kh, (((2,), (2,)), ((0,), (0,))),
                preferred_element_type=jnp.float32) * SCALE
            m = jnp.max(scores, axis=-1, keepdims=True)
            w = jnp.exp(scores - m)
            p = (w / jnp.sum(w, axis=-1, keepdims=True)).astype(jnp.bfloat16)
            ctx3 = lax.dot_general(
                p, vh, (((2,), (1,)), ((0,), (0,))),
                preferred_element_type=jnp.float32)
            ctx = ctx3.reshape(sq, DH).astype(jnp.bfloat16)
            return acc + mm(ctx, wo_g, ((1,), (1,)))

        acc = lax.fori_loop(0, HPD, head_body, acc)

        if h < N_DEV - 1:
            rdma.wait()

    out_ref[...] = acc


def kernel(x, Wq, K_ext, V_ext, Wo):
    bf = jnp.bfloat16
    xg = _group_rows(x[0]).astype(bf)
    wqh = Wq.astype(bf).reshape(Wq.shape[0], HPD, DH).transpose(1, 0, 2)
    woT = Wo.astype(bf).reshape(HPD, DH, Wo.shape[1]).transpose(0, 2, 1)
    w_pack = jnp.stack([wqh, woT], axis=1).reshape(2 * HPD, Wq.shape[0], DH)
    kg = jax.vmap(_group_rows)(K_ext[0].astype(bf).transpose(1, 0, 2))
    vg = jax.vmap(_group_rows)(V_ext[0].astype(bf).transpose(1, 0, 2))

    sq = xg.shape[0]
    skv = kg.shape[1]
    out = pl.pallas_call(
        _body,
        out_shape=jax.ShapeDtypeStruct((sq, Wo.shape[1]), jnp.float32),
        in_specs=[
            pl.BlockSpec(memory_space=pltpu.VMEM),
            pl.BlockSpec(memory_space=pltpu.MemorySpace.HBM),
            pl.BlockSpec(memory_space=pltpu.MemorySpace.HBM),
            pl.BlockSpec(memory_space=pltpu.VMEM),
        ],
        out_specs=pl.BlockSpec(memory_space=pltpu.VMEM),
        scratch_shapes=[
            pltpu.VMEM((2, 2 * HPD, Wq.shape[0], DH), bf),
            pltpu.VMEM((2, HPD, skv, DH), bf),
            pltpu.VMEM((2, HPD, skv, DH), bf),
            pltpu.SemaphoreType.DMA((2,)),
            pltpu.SemaphoreType.DMA((2,)),
            pltpu.SemaphoreType.DMA((2,)),
            pltpu.SemaphoreType.DMA((2,)),
        ],
        compiler_params=pltpu.CompilerParams(
            collective_id=0,
            vmem_limit_bytes=63 * 1024 * 1024,
        ),
    )(xg, kg, vg, w_pack)
    out = out.reshape(NG, sq // 256, 64, -1).transpose(1, 0, 2, 3)
    return out.reshape(sq, -1)[None]


# device time: 411706 ns/iter; 1.1532x vs baseline; 1.1532x over previous
import jax
import jax.numpy as jnp
from jax import lax
from jax.experimental import pallas as pl
from jax.experimental.pallas import tpu as pltpu

N_DEV = 8
HPD = 8
DH = 128
SCALE = 0.08838834764831843


def _body(x_ref, k_ref, v_ref, w_ref, out_ref,
          cw, ccw, kbuf, vbuf,
          cw_send, cw_recv, ccw_send, ccw_recv, kcp_sems, vcp_sems):
    my = lax.axis_index("i")
    left = jnp.mod(my - 1, N_DEV)
    right = jnp.mod(my + 1, N_DEV)

    js = [jnp.mod(my + d, N_DEV)
          for d in (0, -1, 1, -2, 2, -3, 3, 4)]

    def kv_copies(s):
        slot = s % 2
        kcp = pltpu.make_async_copy(
            k_ref.at[pl.ds(js[s] * HPD, HPD)], kbuf.at[slot],
            kcp_sems.at[slot])
        vcp = pltpu.make_async_copy(
            v_ref.at[pl.ds(js[s] * HPD, HPD)], vbuf.at[slot],
            vcp_sems.at[slot])
        return kcp, vcp

    for cp in kv_copies(0):
        cp.start()

    barrier = pltpu.get_barrier_semaphore()
    for nbr in (left, right):
        pl.semaphore_signal(barrier, inc=1, device_id=(nbr,),
                            device_id_type=pl.DeviceIdType.MESH)
    pl.semaphore_wait(barrier, 2)

    cw[0] = w_ref[...]
    ccw[0] = w_ref[...]

    x_bf = x_ref[...]
    sq = x_bf.shape[0]
    skv = kbuf.shape[2]
    row = lax.broadcasted_iota(jnp.int32, (sq, skv), 0)
    col = lax.broadcasted_iota(jnp.int32, (sq, skv), 1)
    mask = ((row // 64) % 4) == ((col // 64) % 4)

    def mm(a, b, dims):
        return lax.dot_general(a, b, dimension_numbers=(dims, ((), ())),
                               preferred_element_type=jnp.float32)

    def head_loop(acc, comm, slot, kv_slot, g_lo, g_hi):
        def head_body(g, acc):
            wq_g = comm[slot, 2 * g]
            wo_g = comm[slot, 2 * g + 1]
            kh = kbuf[kv_slot, g]
            vh = vbuf[kv_slot, g]
            qh = mm(x_bf, wq_g, ((1,), (0,))).astype(jnp.bfloat16)
            scores = mm(qh, kh, ((1,), (1,))) * SCALE
            scores = jnp.where(mask, scores, -1e9)
            m = jnp.max(scores, axis=-1, keepdims=True)
            w = jnp.exp(scores - m)
            p = (w / jnp.sum(w, axis=-1, keepdims=True)).astype(jnp.bfloat16)
            ctx = mm(p, vh, ((1,), (0,))).astype(jnp.bfloat16)
            return acc + mm(ctx, wo_g, ((1,), (1,)))
        return lax.fori_loop(g_lo, g_hi, head_body, acc)

    def make_rdmas(h):
        src = slice(None) if h < 3 else pl.ds(0, HPD)
        ccw_src = slice(None) if h < 3 else pl.ds(HPD, HPD)
        s_slot, r_slot = h % 2, (h + 1) % 2
        r_cw = pltpu.make_async_remote_copy(
            src_ref=cw.at[s_slot, src],
            dst_ref=cw.at[r_slot, src],
            send_sem=cw_send.at[s_slot], recv_sem=cw_recv.at[r_slot],
            device_id=(right,), device_id_type=pl.DeviceIdType.MESH)
        r_ccw = pltpu.make_async_remote_copy(
            src_ref=ccw.at[s_slot, ccw_src],
            dst_ref=ccw.at[r_slot, ccw_src],
            send_sem=ccw_send.at[s_slot], recv_sem=ccw_recv.at[r_slot],
            device_id=(left,), device_id_type=pl.DeviceIdType.MESH)
        return r_cw, r_ccw

    acc = jnp.zeros(out_ref.shape, jnp.float32)

    rdmas = make_rdmas(0)
    for r in rdmas:
        r.start()
    for cp in kv_copies(1):
        cp.start()
    for cp in kv_copies(0):
        cp.wait()
    acc = head_loop(acc, cw, 0, 0, 0, HPD)
    for r in rdmas:
        r.wait()

    for h in range(1, 4):
        slot = h % 2
        rdmas = make_rdmas(h)
        for r in rdmas:
            r.start()
        s_cw, s_ccw = 2 * h - 1, 2 * h
        for cp in kv_copies(s_cw + 1):
            cp.start()
        for cp in kv_copies(s_cw):
            cp.wait()
        acc = head_loop(acc, cw, slot, s_cw % 2, 0, HPD)
        if s_ccw + 1 < 8:
            for cp in kv_copies(s_ccw + 1):
                cp.start()
        for cp in kv_copies(s_ccw):
            cp.wait()
        acc = head_loop(acc, ccw, slot, s_ccw % 2, 0, HPD)
        for r in rdmas:
            r.wait()

    for cp in kv_copies(7):
        cp.wait()
    acc = head_loop(acc, cw, 0, 1, 0, HPD // 2)
    acc = head_loop(acc, ccw, 0, 1, HPD // 2, HPD)

    out_ref[...] = acc


def kernel(x, Wq, K_ext, V_ext, Wo):
    bf = jnp.bfloat16
    xb = x[0].astype(bf)
    wqh = Wq.astype(bf).reshape(Wq.shape[0], HPD, DH).transpose(1, 0, 2)
    woT = Wo.astype(bf).reshape(HPD, DH, Wo.shape[1]).transpose(0, 2, 1)
    w_pack = jnp.stack([wqh, woT], axis=1).reshape(2 * HPD, Wq.shape[0], DH)
    kb = K_ext[0].astype(bf).transpose(1, 0, 2)
    vb = V_ext[0].astype(bf).transpose(1, 0, 2)

    sq = xb.shape[0]
    skv = kb.shape[1]
    out = pl.pallas_call(
        _body,
        out_shape=jax.ShapeDtypeStruct((sq, Wo.shape[1]), jnp.float32),
        in_specs=[
            pl.BlockSpec(memory_space=pltpu.VMEM),
            pl.BlockSpec(memory_space=pltpu.MemorySpace.HBM),
            pl.BlockSpec(memory_space=pltpu.MemorySpace.HBM),
            pl.BlockSpec(memory_space=pltpu.VMEM),
        ],
        out_specs=pl.BlockSpec(memory_space=pltpu.VMEM),
        scratch_shapes=[
            pltpu.VMEM((2, 2 * HPD, Wq.shape[0], DH), bf),
            pltpu.VMEM((2, 2 * HPD, Wq.shape[0], DH), bf),
            pltpu.VMEM((2, HPD, skv, DH), bf),
            pltpu.VMEM((2, HPD, skv, DH), bf),
            pltpu.SemaphoreType.DMA((2,)),
            pltpu.SemaphoreType.DMA((2,)),
            pltpu.SemaphoreType.DMA((2,)),
            pltpu.SemaphoreType.DMA((2,)),
            pltpu.SemaphoreType.DMA((2,)),
            pltpu.SemaphoreType.DMA((2,)),
        ],
        compiler_params=pltpu.CompilerParams(
            collective_id=0,
            vmem_limit_bytes=63 * 1024 * 1024,
        ),
    )(xb, kb, vb, w_pack)
    return out[None]


# device time: 376145 ns/iter; 1.2622x vs baseline; 1.0945x over previous
import jax
import jax.numpy as jnp
from jax import lax
from jax.experimental import pallas as pl
from jax.experimental.pallas import tpu as pltpu

N_DEV = 8
HPD = 8
DH = 128
NG = 4
SCALE = 0.08838834764831843


def _group_rows(a):
    n, d = a.shape
    return a.reshape(n // 256, NG, 64, d).transpose(1, 0, 2, 3).reshape(n, d)


def _body(x_ref, k_ref, v_ref, w_ref, out_ref,
          cw, ccw, kbuf, vbuf,
          cw_send, cw_recv, ccw_send, ccw_recv, kcp_sems, vcp_sems):
    my = lax.axis_index("i")
    left = jnp.mod(my - 1, N_DEV)
    right = jnp.mod(my + 1, N_DEV)

    js = [jnp.mod(my + d, N_DEV)
          for d in (0, -1, 1, -2, 2, -3, 3, 4)]

    def kv_copies(s):
        slot = s % 2
        kcp = pltpu.make_async_copy(
            k_ref.at[pl.ds(js[s] * HPD, HPD)], kbuf.at[slot],
            kcp_sems.at[slot])
        vcp = pltpu.make_async_copy(
            v_ref.at[pl.ds(js[s] * HPD, HPD)], vbuf.at[slot],
            vcp_sems.at[slot])
        return kcp, vcp

    for cp in kv_copies(0):
        cp.start()

    barrier = pltpu.get_barrier_semaphore()
    for nbr in (left, right):
        pl.semaphore_signal(barrier, inc=1, device_id=(nbr,),
                            device_id_type=pl.DeviceIdType.MESH)
    pl.semaphore_wait(barrier, 2)

    cw[0] = w_ref[...]
    ccw[0] = w_ref[...]

    x_bf = x_ref[...]
    sq = x_bf.shape[0]
    gq = sq // NG
    gk = kbuf.shape[2] // NG

    def mm(a, b, dims):
        return lax.dot_general(a, b, dimension_numbers=(dims, ((), ())),
                               preferred_element_type=jnp.float32)

    def head_loop(acc, comm, slot, kv_slot, g_lo, g_hi):
        def head_body(g, acc):
            wq_g = comm[slot, 2 * g]
            wo_g = comm[slot, 2 * g + 1]
            kh = kbuf[kv_slot, g]
            vh = vbuf[kv_slot, g]
            qh = mm(x_bf, wq_g, ((1,), (0,))).astype(jnp.bfloat16)
            ctx = []
            for r in range(NG):
                qr = qh[r * gq:(r + 1) * gq]
                kr = kh[r * gk:(r + 1) * gk]
                vr = vh[r * gk:(r + 1) * gk]
                w = jnp.exp(mm(qr, kr, ((1,), (1,))) * SCALE)
                p = (w / jnp.sum(w, axis=-1, keepdims=True)).astype(jnp.bfloat16)
                ctx.append(mm(p, vr, ((1,), (0,))).astype(jnp.bfloat16))
            ctx = jnp.concatenate(ctx, axis=0)
            return acc + mm(ctx, wo_g, ((1,), (1,)))
        return lax.fori_loop(g_lo, g_hi, head_body, acc)

    def make_rdmas(h):
        src = slice(None) if h < 3 else pl.ds(0, HPD)
        ccw_src = slice(None) if h < 3 else pl.ds(HPD, HPD)
        s_slot, r_slot = h % 2, (h + 1) % 2
        r_cw = pltpu.make_async_remote_copy(
            src_ref=cw.at[s_slot, src],
            dst_ref=cw.at[r_slot, src],
            send_sem=cw_send.at[s_slot], recv_sem=cw_recv.at[r_slot],
            device_id=(right,), device_id_type=pl.DeviceIdType.MESH)
        r_ccw = pltpu.make_async_remote_copy(
            src_ref=ccw.at[s_slot, ccw_src],
            dst_ref=ccw.at[r_slot, ccw_src],
            send_sem=ccw_send.at[s_slot], recv_sem=ccw_recv.at[r_slot],
            device_id=(left,), device_id_type=pl.DeviceIdType.MESH)
        return r_cw, r_ccw

    acc = jnp.zeros(out_ref.shape, jnp.float32)

    rdmas = make_rdmas(0)
    for r in rdmas:
        r.start()
    for cp in kv_copies(1):
        cp.start()
    for cp in kv_copies(0):
        cp.wait()
    acc = head_loop(acc, cw, 0, 0, 0, HPD)
    for r in rdmas:
        r.wait()

    for h in range(1, 4):
        slot = h % 2
        rdmas = make_rdmas(h)
        for r in rdmas:
            r.start()
        s_cw, s_ccw = 2 * h - 1, 2 * h
        for cp in kv_copies(s_cw + 1):
            cp.start()
        for cp in kv_copies(s_cw):
            cp.wait()
        acc = head_loop(acc, cw, slot, s_cw % 2, 0, HPD)
        if s_ccw + 1 < 8:
            for cp in kv_copies(s_ccw + 1):
                cp.start()
        for cp in kv_copies(s_ccw):
            cp.wait()
        acc = head_loop(acc, ccw, slot, s_ccw % 2, 0, HPD)
        for r in rdmas:
            r.wait()

    for cp in kv_copies(7):
        cp.wait()
    acc = head_loop(acc, cw, 0, 1, 0, HPD // 2)
    acc = head_loop(acc, ccw, 0, 1, HPD // 2, HPD)

    out_ref[...] = acc


def kernel(x, Wq, K_ext, V_ext, Wo):
    bf = jnp.bfloat16
    xg = _group_rows(x[0]).astype(bf)
    wqh = Wq.astype(bf).reshape(Wq.shape[0], HPD, DH).transpose(1, 0, 2)
    woT = Wo.astype(bf).reshape(HPD, DH, Wo.shape[1]).transpose(0, 2, 1)
    w_pack = jnp.stack([wqh, woT], axis=1).reshape(2 * HPD, Wq.shape[0], DH)
    kg = jax.vmap(_group_rows)(K_ext[0].astype(bf).transpose(1, 0, 2))
    vg = jax.vmap(_group_rows)(V_ext[0].astype(bf).transpose(1, 0, 2))

    sq = xg.shape[0]
    skv = kg.shape[1]
    out = pl.pallas_call(
        _body,
        out_shape=jax.ShapeDtypeStruct((sq, Wo.shape[1]), jnp.float32),
        in_specs=[
            pl.BlockSpec(memory_space=pltpu.VMEM),
            pl.BlockSpec(memory_space=pltpu.MemorySpace.HBM),
            pl.BlockSpec(memory_space=pltpu.MemorySpace.HBM),
            pl.BlockSpec(memory_space=pltpu.VMEM),
        ],
        out_specs=pl.BlockSpec(memory_space=pltpu.VMEM),
        scratch_shapes=[
            pltpu.VMEM((2, 2 * HPD, Wq.shape[0], DH), bf),
            pltpu.VMEM((2, 2 * HPD, Wq.shape[0], DH), bf),
            pltpu.VMEM((2, HPD, skv, DH), bf),
            pltpu.VMEM((2, HPD, skv, DH), bf),
            pltpu.SemaphoreType.DMA((2,)),
            pltpu.SemaphoreType.DMA((2,)),
            pltpu.SemaphoreType.DMA((2,)),
            pltpu.SemaphoreType.DMA((2,)),
            pltpu.SemaphoreType.DMA((2,)),
            pltpu.SemaphoreType.DMA((2,)),
        ],
        compiler_params=pltpu.CompilerParams(
            collective_id=0,
            vmem_limit_bytes=63 * 1024 * 1024,
        ),
    )(xg, kg, vg, w_pack)
    out = out.reshape(NG, sq // 256, 64, -1).transpose(1, 0, 2, 3)
    return out.reshape(sq, -1)[None]


# device time: 300884 ns/iter; 1.5779x vs baseline; 1.2501x over previous
import jax
import jax.numpy as jnp
from jax import lax
from jax.experimental import pallas as pl
from jax.experimental.pallas import tpu as pltpu

N_DEV = 8
HPD = 8
DH = 128
NG = 4
SCALE = 0.08838834764831843


def _group_rows(a):
    n, d = a.shape
    return a.reshape(n // 256, NG, 64, d).transpose(1, 0, 2, 3).reshape(n, d)


def _body(x_ref, k_ref, v_ref, w_ref, out_ref,
          cw, ccw, kbuf, vbuf, q_scr, ctx_scr,
          cw_s1, cw_r1, ccw_s1, ccw_r1,
          cw_s2, cw_r2, ccw_s2, ccw_r2, kcp_sems, vcp_sems):
    my = lax.axis_index("i")
    left = jnp.mod(my - 1, N_DEV)
    right = jnp.mod(my + 1, N_DEV)

    js = [jnp.mod(my + d, N_DEV)
          for d in (0, -1, 1, -2, 2, -3, 3, 4)]

    def kv_copies(s):
        slot = s % 2
        kcp = pltpu.make_async_copy(
            k_ref.at[pl.ds(js[s] * HPD, HPD)], kbuf.at[slot],
            kcp_sems.at[slot])
        vcp = pltpu.make_async_copy(
            v_ref.at[pl.ds(js[s] * HPD, HPD)], vbuf.at[slot],
            vcp_sems.at[slot])
        return kcp, vcp

    for cp in kv_copies(0):
        cp.start()

    barrier = pltpu.get_barrier_semaphore()
    for nbr in (left, right):
        pl.semaphore_signal(barrier, inc=1, device_id=(nbr,),
                            device_id_type=pl.DeviceIdType.MESH)
    pl.semaphore_wait(barrier, 2)

    cw[0] = w_ref[...]
    ccw[0] = w_ref[...]

    x_bf = x_ref[...]
    sq = x_bf.shape[0]
    gq = sq // NG
    gk = kbuf.shape[2] // NG

    def mm(a, b, dims):
        return lax.dot_general(a, b, dimension_numbers=(dims, ((), ())),
                               preferred_element_type=jnp.float32)

    def compute_block(acc, comm, slot, kv_slot, g_lo, g_hi):
        nh = g_hi - g_lo
        wq_blk = jnp.concatenate(
            [comm[slot, gg] for gg in range(g_lo, g_hi)], axis=1)
        q_all = mm(x_bf, wq_blk, ((1,), (0,))).astype(jnp.bfloat16)
        for i in range(nh):
            q_scr[g_lo + i] = q_all[:, i * DH:(i + 1) * DH]

        def head_body(g, c):
            qh = q_scr[g]
            kh = kbuf[kv_slot, g]
            vh = vbuf[kv_slot, g]
            sc = jnp.concatenate(
                [mm(qh[r * gq:(r + 1) * gq], kh[r * gk:(r + 1) * gk],
                    ((1,), (1,))) for r in range(NG)], axis=0)
            w = jnp.exp(sc * SCALE)
            p = (w / jnp.sum(w, axis=-1, keepdims=True)).astype(jnp.bfloat16)
            ctx_scr[g] = jnp.concatenate(
                [mm(p[r * gq:(r + 1) * gq], vh[r * gk:(r + 1) * gk],
                    ((1,), (0,))) for r in range(NG)],
                axis=0).astype(jnp.bfloat16)
            return c
        lax.fori_loop(g_lo, g_hi, head_body, 0)

        ctx_all = jnp.concatenate(
            [ctx_scr[gg] for gg in range(g_lo, g_hi)], axis=1)
        wo_blk = jnp.concatenate(
            [comm[slot, HPD + gg] for gg in range(g_lo, g_hi)], axis=1)
        return acc + mm(ctx_all, wo_blk, ((1,), (1,)))

    def make_rdmas(h):
        s, r = h % 2, (h + 1) % 2

        def rd(comm, src, sem_s, sem_r, dev):
            return pltpu.make_async_remote_copy(
                src_ref=comm.at[s, src], dst_ref=comm.at[r, src],
                send_sem=sem_s.at[s], recv_sem=sem_r.at[r],
                device_id=(dev,), device_id_type=pl.DeviceIdType.MESH)

        if h < 3:
            return [rd(cw, slice(None), cw_s1, cw_r1, right),
                    rd(ccw, slice(None), ccw_s1, ccw_r1, left)]
        half = HPD // 2
        return [rd(cw, pl.ds(0, half), cw_s1, cw_r1, right),
                rd(cw, pl.ds(HPD, half), cw_s2, cw_r2, right),
                rd(ccw, pl.ds(half, half), ccw_s1, ccw_r1, left),
                rd(ccw, pl.ds(HPD + half, half), ccw_s2, ccw_r2, left)]

    acc = jnp.zeros(out_ref.shape, jnp.float32)

    rdmas = make_rdmas(0)
    for r in rdmas:
        r.start()
    for cp in kv_copies(1):
        cp.start()
    for cp in kv_copies(0):
        cp.wait()
    acc = compute_block(acc, cw, 0, 0, 0, HPD)
    for r in rdmas:
        r.wait()

    for h in range(1, 4):
        slot = h % 2
        rdmas = make_rdmas(h)
        for r in rdmas:
            r.start()
        s_cw, s_ccw = 2 * h - 1, 2 * h
        for cp in kv_copies(s_cw + 1):
            cp.start()
        for cp in kv_copies(s_cw):
            cp.wait()
        acc = compute_block(acc, cw, slot, s_cw % 2, 0, HPD)
        if s_ccw + 1 < 8:
            for cp in kv_copies(s_ccw + 1):
                cp.start()
        for cp in kv_copies(s_ccw):
            cp.wait()
        acc = compute_block(acc, ccw, slot, s_ccw % 2, 0, HPD)
        for r in rdmas:
            r.wait()

    for cp in kv_copies(7):
        cp.wait()
    acc = compute_block(acc, cw, 0, 1, 0, HPD // 2)
    acc = compute_block(acc, ccw, 0, 1, HPD // 2, HPD)

    out_ref[...] = acc


def kernel(x, Wq, K_ext, V_ext, Wo):
    bf = jnp.bfloat16
    xg = _group_rows(x[0]).astype(bf)
    wqh = Wq.astype(bf).reshape(Wq.shape[0], HPD, DH).transpose(1, 0, 2)
    woT = Wo.astype(bf).reshape(HPD, DH, Wo.shape[1]).transpose(0, 2, 1)
    w_pack = jnp.concatenate([wqh, woT], axis=0)
    kg = jax.vmap(_group_rows)(K_ext[0].astype(bf).transpose(1, 0, 2))
    vg = jax.vmap(_group_rows)(V_ext[0].astype(bf).transpose(1, 0, 2))

    sq = xg.shape[0]
    skv = kg.shape[1]
    out = pl.pallas_call(
        _body,
        out_shape=jax.ShapeDtypeStruct((sq, Wo.shape[1]), jnp.float32),
        in_specs=[
            pl.BlockSpec(memory_space=pltpu.VMEM),
            pl.BlockSpec(memory_space=pltpu.MemorySpace.HBM),
            pl.BlockSpec(memory_space=pltpu.MemorySpace.HBM),
            pl.BlockSpec(memory_space=pltpu.VMEM),
        ],
        out_specs=pl.BlockSpec(memory_space=pltpu.VMEM),
        scratch_shapes=[
            pltpu.VMEM((2, 2 * HPD, Wq.shape[0], DH), bf),
            pltpu.VMEM((2, 2 * HPD, Wq.shape[0], DH), bf),
            pltpu.VMEM((2, HPD, skv, DH), bf),
            pltpu.VMEM((2, HPD, skv, DH), bf),
            pltpu.VMEM((HPD, 1024, DH), bf),
            pltpu.VMEM((HPD, 1024, DH), bf),
            pltpu.SemaphoreType.DMA((2,)),
            pltpu.SemaphoreType.DMA((2,)),
            pltpu.SemaphoreType.DMA((2,)),
            pltpu.SemaphoreType.DMA((2,)),
            pltpu.SemaphoreType.DMA((2,)),
            pltpu.SemaphoreType.DMA((2,)),
            pltpu.SemaphoreType.DMA((2,)),
            pltpu.SemaphoreType.DMA((2,)),
            pltpu.SemaphoreType.DMA((2,)),
            pltpu.SemaphoreType.DMA((2,)),
        ],
        compiler_params=pltpu.CompilerParams(
            collective_id=0,
            vmem_limit_bytes=63 * 1024 * 1024,
        ),
    )(xg, kg, vg, w_pack)
    out = out.reshape(NG, sq // 256, 64, -1).transpose(1, 0, 2, 3)
    return out.reshape(sq, -1)[None]


# device time: 296181 ns/iter; 1.6030x vs baseline; 1.0159x over previous
import jax
import jax.numpy as jnp
from jax import lax
from jax.experimental import pallas as pl
from jax.experimental.pallas import tpu as pltpu

N_DEV = 8
HPD = 8
DH = 128
NG = 4
SCALE = 0.08838834764831843


def _group_rows(a):
    n, d = a.shape
    return a.reshape(n // 256, NG, 64, d).transpose(1, 0, 2, 3).reshape(n, d)


def _body(x_ref, k_ref, v_ref, wq_ref, wo_ref, out_ref,
          cw_wq, cw_wo, ccw_wq, ccw_wo, kbuf, vbuf, q_scr, ctx_scr,
          cw_s1, cw_r1, ccw_s1, ccw_r1,
          cw_s2, cw_r2, ccw_s2, ccw_r2, kcp_sems, vcp_sems):
    my = lax.axis_index("i")
    left = jnp.mod(my - 1, N_DEV)
    right = jnp.mod(my + 1, N_DEV)

    js = [jnp.mod(my + d, N_DEV)
          for d in (0, -1, 1, -2, 2, -3, 3, 4)]

    def kv_copies(s):
        slot = s % 2
        kcp = pltpu.make_async_copy(
            k_ref.at[pl.ds(js[s] * HPD, HPD)], kbuf.at[slot],
            kcp_sems.at[slot])
        vcp = pltpu.make_async_copy(
            v_ref.at[pl.ds(js[s] * HPD, HPD)], vbuf.at[slot],
            vcp_sems.at[slot])
        return kcp, vcp

    for cp in kv_copies(0):
        cp.start()

    barrier = pltpu.get_barrier_semaphore()
    for nbr in (left, right):
        pl.semaphore_signal(barrier, inc=1, device_id=(nbr,),
                            device_id_type=pl.DeviceIdType.MESH)
    pl.semaphore_wait(barrier, 2)

    for g in range(HPD):
        cw_wq[0, g] = wq_ref[:, g * DH:(g + 1) * DH]
        ccw_wq[0, g] = wq_ref[:, g * DH:(g + 1) * DH]
    cw_wo[0] = wo_ref[...]
    ccw_wo[0] = wo_ref[...]

    sq = x_ref.shape[0]
    gq = sq // NG
    gk = kbuf.shape[2] // NG

    x_bf = jnp.concatenate(
        [x_ref[(hi * NG + r) * 64:(hi * NG + r + 1) * 64]
         for r in range(NG) for hi in range(NG)], axis=0)

    def mm(a, b, dims):
        return lax.dot_general(a, b, dimension_numbers=(dims, ((), ())),
                               preferred_element_type=jnp.float32)

    def compute_block(acc, comm_wq, comm_wo, slot, kv_slot, g_lo, g_hi):
        nh = g_hi - g_lo
        wq_blk = jnp.concatenate(
            [comm_wq[slot, gg] for gg in range(g_lo, g_hi)], axis=1)
        q_all = mm(x_bf, wq_blk, ((1,), (0,))).astype(jnp.bfloat16)
        for i in range(nh):
            q_scr[g_lo + i] = q_all[:, i * DH:(i + 1) * DH]

        def head_body(g, c):
            qh = q_scr[g]
            kh = kbuf[kv_slot, g]
            vh = vbuf[kv_slot, g]
            sc = jnp.concatenate(
                [mm(qh[r * gq:(r + 1) * gq], kh[r * gk:(r + 1) * gk],
                    ((1,), (1,))) for r in range(NG)], axis=0)
            w = jnp.exp(sc * SCALE)
            p = (w / jnp.sum(w, axis=-1, keepdims=True)).astype(jnp.bfloat16)
            ctx_scr[g] = jnp.concatenate(
                [mm(p[r * gq:(r + 1) * gq], vh[r * gk:(r + 1) * gk],
                    ((1,), (0,))) for r in range(NG)],
                axis=0).astype(jnp.bfloat16)
            return c
        lax.fori_loop(g_lo, g_hi, head_body, 0)

        ctx_all = jnp.concatenate(
            [ctx_scr[gg] for gg in range(g_lo, g_hi)], axis=1)
        wo_flat = comm_wo[slot, g_lo:g_hi].reshape(nh * DH, -1)
        return acc + mm(ctx_all, wo_flat, ((1,), (0,)))

    def make_rdmas(h):
        s, r = h % 2, (h + 1) % 2
        cw_sl = slice(None) if h < 3 else pl.ds(0, HPD // 2)
        ccw_sl = slice(None) if h < 3 else pl.ds(HPD // 2, HPD // 2)

        def rd(comm, src, sem_s, sem_r, dev):
            return pltpu.make_async_remote_copy(
                src_ref=comm.at[s, src], dst_ref=comm.at[r, src],
                send_sem=sem_s.at[s], recv_sem=sem_r.at[r],
                device_id=(dev,), device_id_type=pl.DeviceIdType.MESH)

        return [rd(cw_wq, cw_sl, cw_s1, cw_r1, right),
                rd(cw_wo, cw_sl, cw_s2, cw_r2, right),
                rd(ccw_wq, ccw_sl, ccw_s1, ccw_r1, left),
                rd(ccw_wo, ccw_sl, ccw_s2, ccw_r2, left)]

    acc = jnp.zeros(out_ref.shape, jnp.float32)

    rdmas = make_rdmas(0)
    for r in rdmas:
        r.start()
    for cp in kv_copies(1):
        cp.start()
    for cp in kv_copies(0):
        cp.wait()
    acc = compute_block(acc, cw_wq, cw_wo, 0, 0, 0, HPD)
    for r in rdmas:
        r.wait()

    for h in range(1, 4):
        slot = h % 2
        rdmas = make_rdmas(h)
        for r in rdmas:
            r.start()
        s_cw, s_ccw = 2 * h - 1, 2 * h
        for cp in kv_copies(s_cw + 1):
            cp.start()
        for cp in kv_copies(s_cw):
            cp.wait()
        acc = compute_block(acc, cw_wq, cw_wo, slot, s_cw % 2, 0, HPD)
        if s_ccw + 1 < 8:
            for cp in kv_copies(s_ccw + 1):
                cp.start()
        for cp in kv_copies(s_ccw):
            cp.wait()
        acc = compute_block(acc, ccw_wq, ccw_wo, slot, s_ccw % 2, 0, HPD)
        for r in rdmas:
            r.wait()

    for cp in kv_copies(7):
        cp.wait()
    acc = compute_block(acc, cw_wq, cw_wo, 0, 1, 0, HPD // 2)
    acc = compute_block(acc, ccw_wq, ccw_wo, 0, 1, HPD // 2, HPD)

    for r in range(NG):
        for hi in range(NG):
            out_ref[(hi * NG + r) * 64:(hi * NG + r + 1) * 64] = (
                acc[r * NG * 64 + hi * 64:r * NG * 64 + (hi + 1) * 64])


def kernel(x, Wq, K_ext, V_ext, Wo):
    bf = jnp.bfloat16
    xb = x[0].astype(bf)
    wq = Wq.astype(bf)
    wo = Wo.astype(bf).reshape(HPD, DH, Wo.shape[1])
    kg = jax.vmap(_group_rows)(K_ext[0].astype(bf).transpose(1, 0, 2))
    vg = jax.vmap(_group_rows)(V_ext[0].astype(bf).transpose(1, 0, 2))

    sq = xb.shape[0]
    skv = kg.shape[1]
    n_out = Wo.shape[1]
    out = pl.pallas_call(
        _body,
        out_shape=jax.ShapeDtypeStruct((sq, n_out), jnp.float32),
        in_specs=[
            pl.BlockSpec(memory_space=pltpu.VMEM),
            pl.BlockSpec(memory_space=pltpu.MemorySpace.HBM),
            pl.BlockSpec(memory_space=pltpu.MemorySpace.HBM),
            pl.BlockSpec(memory_space=pltpu.VMEM),
            pl.BlockSpec(memory_space=pltpu.VMEM),
        ],
        out_specs=pl.BlockSpec(memory_space=pltpu.VMEM),
        scratch_shapes=[
            pltpu.VMEM((2, HPD, sq, DH), bf),
            pltpu.VMEM((2, HPD, DH, n_out), bf),
            pltpu.VMEM((2, HPD, sq, DH), bf),
            pltpu.VMEM((2, HPD, DH, n_out), bf),
            pltpu.VMEM((2, HPD, skv, DH), bf),
            pltpu.VMEM((2, HPD, skv, DH), bf),
            pltpu.VMEM((HPD, sq, DH), bf),
            pltpu.VMEM((HPD, sq, DH), bf),
            pltpu.SemaphoreType.DMA((2,)),
            pltpu.SemaphoreType.DMA((2,)),
            pltpu.SemaphoreType.DMA((2,)),
            pltpu.SemaphoreType.DMA((2,)),
            pltpu.SemaphoreType.DMA((2,)),
            pltpu.SemaphoreType.DMA((2,)),
            pltpu.SemaphoreType.DMA((2,)),
            pltpu.SemaphoreType.DMA((2,)),
            pltpu.SemaphoreType.DMA((2,)),
            pltpu.SemaphoreType.DMA((2,)),
        ],
        compiler_params=pltpu.CompilerParams(
            collective_id=0,
            vmem_limit_bytes=63 * 1024 * 1024,
        ),
    )(xb, kg, vg, wq, wo)
    return out[None]


# device time: 198951 ns/iter; 2.3864x vs baseline; 1.4887x over previous
import jax
import jax.numpy as jnp
from jax import lax
from jax.experimental import pallas as pl
from jax.experimental.pallas import tpu as pltpu

N_DEV = 8
HPD = 8
DH = 128
NG = 4
SCALE = 0.08838834764831843


def _group_rows(a):
    n, d = a.shape
    return a.reshape(n // 256, NG, 64, d).transpose(1, 0, 2, 3).reshape(n, d)


def _body(x_ref, k_ref, v_ref, wq_ref, wo_ref, out_ref,
          cw_wq, cw_wo, ccw_wq, ccw_wo, kbuf, vbuf, q_scr, ctx_scr,
          cw_s1, cw_r1, ccw_s1, ccw_r1,
          cw_s2, cw_r2, ccw_s2, ccw_r2, kcp_sems, vcp_sems):
    my = lax.axis_index("i")
    left = jnp.mod(my - 1, N_DEV)
    right = jnp.mod(my + 1, N_DEV)

    js = [jnp.mod(my + d, N_DEV)
          for d in (0, -1, 1, -2, 2, -3, 3, 4)]

    def kv_copies(s):
        slot = s % 2
        cps = []
        for g in range(HPD):
            ga = js[s] * HPD + g
            cps.append(pltpu.make_async_copy(
                k_ref.at[:, :, :, ga, :], kbuf.at[slot, g],
                kcp_sems.at[slot]))
            cps.append(pltpu.make_async_copy(
                v_ref.at[:, :, :, ga, :], vbuf.at[slot, g],
                vcp_sems.at[slot]))
        return cps

    for cp in kv_copies(0):
        cp.start()

    barrier = pltpu.get_barrier_semaphore()
    for nbr in (left, right):
        pl.semaphore_signal(barrier, inc=1, device_id=(nbr,),
                            device_id_type=pl.DeviceIdType.MESH)
    pl.semaphore_wait(barrier, 2)

    for g in range(HPD):
        cw_wq[0, g] = wq_ref[:, g * DH:(g + 1) * DH]
        ccw_wq[0, g] = wq_ref[:, g * DH:(g + 1) * DH]
    cw_wo[0] = wo_ref[...]
    ccw_wo[0] = wo_ref[...]

    sq = x_ref.shape[0]
    gq = sq // NG
    gk = NG * 64

    x_bf = jnp.concatenate(
        [x_ref[(hi * NG + r) * 64:(hi * NG + r + 1) * 64]
         for r in range(NG) for hi in range(NG)], axis=0)

    def mm(a, b, dims):
        return lax.dot_general(a, b, dimension_numbers=(dims, ((), ())),
                               preferred_element_type=jnp.float32)

    def compute_block(acc, comm_wq, comm_wo, slot, kv_slot, g_lo, g_hi):
        nh = g_hi - g_lo
        wq_blk = jnp.concatenate(
            [comm_wq[slot, gg] for gg in range(g_lo, g_hi)], axis=1)
        q_all = (mm(x_bf, wq_blk, ((1,), (0,))) * SCALE).astype(jnp.bfloat16)
        for i in range(nh):
            q_scr[g_lo + i] = q_all[:, i * DH:(i + 1) * DH]

        def head_body(g, c):
            qh = q_scr[g]
            kh = kbuf[kv_slot, g].astype(jnp.bfloat16)
            vh = vbuf[kv_slot, g].astype(jnp.bfloat16)
            sc = jnp.concatenate(
                [mm(qh[r * gq:(r + 1) * gq], kh[:, r].reshape(gk, DH),
                    ((1,), (1,))) for r in range(NG)], axis=0)
            w = jnp.exp(sc)
            recip = 1.0 / jnp.sum(w, axis=-1, keepdims=True)
            p = w.astype(jnp.bfloat16)
            ctx_u = jnp.concatenate(
                [mm(p[r * gq:(r + 1) * gq], vh[:, r].reshape(gk, DH),
                    ((1,), (0,))) for r in range(NG)], axis=0)
            ctx_scr[g] = (ctx_u * recip).astype(jnp.bfloat16)
            return c
        lax.fori_loop(g_lo, g_hi, head_body, 0)

        ctx_all = jnp.concatenate(
            [ctx_scr[gg] for gg in range(g_lo, g_hi)], axis=1)
        wo_flat = comm_wo[slot, g_lo:g_hi].reshape(nh * DH, -1)
        return acc + mm(ctx_all, wo_flat, ((1,), (0,)))

    def make_rdmas(h):
        s, r = h % 2, (h + 1) % 2
        cw_sl = slice(None) if h < 3 else pl.ds(0, HPD // 2)
        ccw_sl = slice(None) if h < 3 else pl.ds(HPD // 2, HPD // 2)

        def rd(comm, src, sem_s, sem_r, dev):
            return pltpu.make_async_remote_copy(
                src_ref=comm.at[s, src], dst_ref=comm.at[r, src],
                send_sem=sem_s.at[s], recv_sem=sem_r.at[r],
                device_id=(dev,), device_id_type=pl.DeviceIdType.MESH)

        return [rd(cw_wq, cw_sl, cw_s1, cw_r1, right),
                rd(cw_wo, cw_sl, cw_s2, cw_r2, right),
                rd(ccw_wq, ccw_sl, ccw_s1, ccw_r1, left),
                rd(ccw_wo, ccw_sl, ccw_s2, ccw_r2, left)]

    acc = jnp.zeros(out_ref.shape, jnp.float32)

    rdmas = make_rdmas(0)
    for r in rdmas:
        r.start()
    for cp in kv_copies(1):
        cp.start()
    for cp in kv_copies(0):
        cp.wait()
    acc = compute_block(acc, cw_wq, cw_wo, 0, 0, 0, HPD)
    for r in rdmas:
        r.wait()

    for h in range(1, 4):
        slot = h % 2
        rdmas = make_rdmas(h)
        for r in rdmas:
            r.start()
        s_cw, s_ccw = 2 * h - 1, 2 * h
        for cp in kv_copies(s_cw + 1):
            cp.start()
        for cp in kv_copies(s_cw):
            cp.wait()
        acc = compute_block(acc, cw_wq, cw_wo, slot, s_cw % 2, 0, HPD)
        if s_ccw + 1 < 8:
            for cp in kv_copies(s_ccw + 1):
                cp.start()
        for cp in kv_copies(s_ccw):
            cp.wait()
        acc = compute_block(acc, ccw_wq, ccw_wo, slot, s_ccw % 2, 0, HPD)
        for r in rdmas:
            r.wait()

    for cp in kv_copies(7):
        cp.wait()
    acc = compute_block(acc, cw_wq, cw_wo, 0, 1, 0, HPD // 2)
    acc = compute_block(acc, ccw_wq, ccw_wo, 0, 1, HPD // 2, HPD)

    for r in range(NG):
        for hi in range(NG):
            out_ref[(hi * NG + r) * 64:(hi * NG + r + 1) * 64] = (
                acc[r * NG * 64 + hi * 64:r * NG * 64 + (hi + 1) * 64])


def kernel(x, Wq, K_ext, V_ext, Wo):
    bf = jnp.bfloat16
    xb = x[0].astype(bf)
    wq = Wq.astype(bf)
    wo = Wo.astype(bf).reshape(HPD, DH, Wo.shape[1])
    skv = K_ext.shape[1]
    kg = K_ext[0].reshape(NG, NG, skv // 16, K_ext.shape[2], DH)
    vg = V_ext[0].reshape(NG, NG, skv // 16, V_ext.shape[2], DH)

    sq = xb.shape[0]
    n_out = Wo.shape[1]
    out = pl.pallas_call(
        _body,
        out_shape=jax.ShapeDtypeStruct((sq, n_out), jnp.float32),
        in_specs=[
            pl.BlockSpec(memory_space=pltpu.VMEM),
            pl.BlockSpec(memory_space=pltpu.MemorySpace.HBM),
            pl.BlockSpec(memory_space=pltpu.MemorySpace.HBM),
            pl.BlockSpec(memory_space=pltpu.VMEM),
            pl.BlockSpec(memory_space=pltpu.VMEM),
        ],
        out_specs=pl.BlockSpec(memory_space=pltpu.VMEM),
        scratch_shapes=[
            pltpu.VMEM((2, HPD, sq, DH), bf),
            pltpu.VMEM((2, HPD, DH, n_out), bf),
            pltpu.VMEM((2, HPD, sq, DH), bf),
            pltpu.VMEM((2, HPD, DH, n_out), bf),
            pltpu.VMEM((2, HPD, NG, NG, skv // 16, DH), jnp.float32),
            pltpu.VMEM((2, HPD, NG, NG, skv // 16, DH), jnp.float32),
            pltpu.VMEM((HPD, sq, DH), bf),
            pltpu.VMEM((HPD, sq, DH), bf),
            pltpu.SemaphoreType.DMA((2,)),
            pltpu.SemaphoreType.DMA((2,)),
            pltpu.SemaphoreType.DMA((2,)),
            pltpu.SemaphoreType.DMA((2,)),
            pltpu.SemaphoreType.DMA((2,)),
            pltpu.SemaphoreType.DMA((2,)),
            pltpu.SemaphoreType.DMA((2,)),
            pltpu.SemaphoreType.DMA((2,)),
            pltpu.SemaphoreType.DMA((2,)),
            pltpu.SemaphoreType.DMA((2,)),
        ],
        compiler_params=pltpu.CompilerParams(
            collective_id=0,
            vmem_limit_bytes=63 * 1024 * 1024,
        ),
    )(xb, kg, vg, wq, wo)
    return out[None]
